# baseline (speedup 1.0000x reference)
import functools

import jax
import jax.numpy as jnp
from jax import lax
from jax.experimental import pallas as pl
from jax.experimental.pallas import tpu as pltpu
from jax.experimental.pallas import tpu_sc as plsc

B, N, L, D = 8, 4, 2048, 768
G = 320
NSUB = 16
BPC = B // 2
ROWS = 8
NGRP = L // ROWS
DV = D // 16


def _sc_pool(x, fidx, wflat, lens):
    mesh = plsc.VectorSubcoreMesh(core_axis_name="c", subcore_axis_name="s")

    @functools.partial(
        pl.kernel,
        out_type=jax.ShapeDtypeStruct((B * D,), jnp.float32),
        mesh=mesh,
        scratch_types=[
            pltpu.VMEM_SHARED((G * G,), jnp.float32),
            pltpu.VMEM_SHARED((NSUB, BPC * D), jnp.float32),
            pltpu.VMEM((G * G,), jnp.float32),
            pltpu.VMEM((ROWS, D), jnp.float32),
            pltpu.VMEM((ROWS, D), jnp.float32),
            pltpu.VMEM((L,), jnp.int32),
            pltpu.VMEM((16,), jnp.int32),
            pltpu.VMEM((D,), jnp.float32),
            pltpu.VMEM((16,), jnp.float32),
            pltpu.SemaphoreType.DMA,
            pltpu.SemaphoreType.DMA,
            pltpu.SemaphoreType.DMA,
            pltpu.SemaphoreType.DMA,
        ],
        compiler_params=pltpu.CompilerParams(needs_layout_passes=False),
    )
    def k(x_hbm, fidx_hbm, wt_hbm, len_hbm, out_hbm,
          table_sh, part_sh, table_v, slab_a, slab_b, fidx_v, len_v,
          acc_v, wg_v, sem_t, sem_a, sem_b, sem_f):
        c = lax.axis_index("c")
        sid = lax.axis_index("s")

        @pl.when(sid == 0)
        def _():
            pltpu.sync_copy(wt_hbm, table_sh)
        pltpu.sync_copy(len_hbm, len_v.at[pl.ds(0, 8)])
        plsc.subcore_barrier()
        cp_t = pltpu.make_async_copy(table_sh, table_v, sem_t)
        cp_t.start()
        zf = jnp.zeros((16,), jnp.float32)
        cp_t.wait()
        iot = lax.iota(jnp.int32, 16)

        def compute(slab, g, lenb_vec):
            ids = jnp.minimum(8 * g + iot, L - 1)
            fv = plsc.load_gather(fidx_v, [ids])
            wg = plsc.load_gather(table_v, [fv])
            pos = 8 * g + iot
            wg = jnp.where(pos < lenb_vec, wg, jnp.zeros_like(wg))
            wg_v[...] = wg

            def row_body(r, cr):
                wsp = plsc.load_gather(
                    wg_v, [jnp.full((16,), r, jnp.int32)])
                for d in range(DV):
                    xv = slab[r, pl.ds(16 * d, 16)]
                    plsc.addupdate(acc_v.at[pl.ds(16 * d, 16)], xv * wsp)
                return cr

            lax.fori_loop(0, ROWS, row_body, 0)

        def batch_body(bb, carry):
            b = c * BPC + bb
            cp_f = pltpu.make_async_copy(
                fidx_hbm.at[pl.ds(b * L, L)], fidx_v, sem_f)
            cp_f.start()
            for d in range(DV):
                acc_v[pl.ds(16 * d, 16)] = zf
            lenb_vec = plsc.load_gather(
                len_v, [jnp.full((16,), b, jnp.int32)])
            lenb = lenb_vec[0]
            gb = (lenb + ROWS - 1) // ROWS
            cnt = jnp.maximum((gb - sid + NSUB - 1) // NSUB, 0)

            def slab_copy(buf, sem, g):
                return pltpu.make_async_copy(
                    x_hbm.at[b, N - 1, pl.ds(8 * g, 8), :], buf, sem)

            @pl.when(cnt > 0)
            def _():
                slab_copy(slab_a, sem_a, sid).start()
            cp_f.wait()

            def pair_body(jj, carry2):
                g0 = sid + NSUB * 2 * jj
                g1 = g0 + NSUB
                g2 = g1 + NSUB

                @pl.when(2 * jj + 1 < cnt)
                def _():
                    slab_copy(slab_b, sem_b, g1).start()

                slab_copy(slab_a, sem_a, g0).wait()
                compute(slab_a, g0, lenb_vec)

                @pl.when(2 * jj + 2 < cnt)
                def _():
                    slab_copy(slab_a, sem_a, g2).start()

                @pl.when(2 * jj + 1 < cnt)
                def _():
                    slab_copy(slab_b, sem_b, g1).wait()
                    compute(slab_b, g1, lenb_vec)

                return carry2

            lax.fori_loop(0, (cnt + 1) // 2, pair_body, 0)
            pltpu.sync_copy(acc_v, part_sh.at[sid, pl.ds(bb * D, D)])
            return carry

        lax.fori_loop(0, BPC, batch_body, 0)

        plsc.subcore_barrier()

        @pl.when(sid < BPC)
        def _():
            bb = sid
            b = c * BPC + bb
            pltpu.sync_copy(
                part_sh.at[pl.ds(0, ROWS), pl.ds(bb * D, D)], slab_a)
            pltpu.sync_copy(
                part_sh.at[pl.ds(ROWS, ROWS), pl.ds(bb * D, D)], slab_b)
            for d in range(DV):
                acc_v[pl.ds(16 * d, 16)] = zf

            def comb_body(r, cr):
                for d in range(DV):
                    plsc.addupdate(
                        acc_v.at[pl.ds(16 * d, 16)],
                        slab_a[r, pl.ds(16 * d, 16)]
                        + slab_b[r, pl.ds(16 * d, 16)])
                return cr

            lax.fori_loop(0, ROWS, comb_body, 0)
            pltpu.sync_copy(acc_v, out_hbm.at[pl.ds(b * D, D)])

    return k(x, fidx, wflat, lens)


def kernel(input_feature, input_lengths, vq_indices, weight):
    lens = input_lengths.astype(jnp.int32)
    fidx = (vq_indices[..., 0] * G + vq_indices[..., 1]).reshape(-1)
    out = _sc_pool(input_feature, fidx, weight.reshape(-1), lens)
    return out.reshape(B, D)


# speedup vs baseline: 1.7554x; 1.7554x over previous
import functools

import jax
import jax.numpy as jnp
from jax import lax
from jax.experimental import pallas as pl
from jax.experimental.pallas import tpu as pltpu
from jax.experimental.pallas import tpu_sc as plsc

B, N, L, D = 8, 4, 2048, 768
G = 320
NSUB = 16
BPC = B // 2
ROWS = 8
NGRP = L // ROWS
DV = D // 16


def _sc_pool(x, fidx, wflat, lens):
    mesh = plsc.VectorSubcoreMesh(core_axis_name="c", subcore_axis_name="s")

    @functools.partial(
        pl.kernel,
        out_type=jax.ShapeDtypeStruct((B * D,), jnp.float32),
        mesh=mesh,
        scratch_types=[
            pltpu.VMEM_SHARED((G * G,), jnp.float32),
            pltpu.VMEM_SHARED((NSUB, BPC * D), jnp.float32),
            pltpu.VMEM((G * G,), jnp.float32),
            pltpu.VMEM((ROWS, D), jnp.float32),
            pltpu.VMEM((ROWS, D), jnp.float32),
            pltpu.VMEM((L,), jnp.int32),
            pltpu.VMEM((16,), jnp.int32),
            pltpu.VMEM((D,), jnp.float32),
            pltpu.VMEM((16,), jnp.float32),
            pltpu.SemaphoreType.DMA,
            pltpu.SemaphoreType.DMA,
            pltpu.SemaphoreType.DMA,
            pltpu.SemaphoreType.DMA,
        ],
        compiler_params=pltpu.CompilerParams(needs_layout_passes=False),
    )
    def k(x_hbm, fidx_hbm, wt_hbm, len_hbm, out_hbm,
          table_sh, part_sh, table_v, slab_a, slab_b, fidx_v, len_v,
          acc_v, wg_v, sem_t, sem_a, sem_b, sem_f):
        c = lax.axis_index("c")
        sid = lax.axis_index("s")

        @pl.when(sid == 0)
        def _():
            pltpu.sync_copy(wt_hbm, table_sh)
        pltpu.sync_copy(len_hbm, len_v.at[pl.ds(0, 8)])
        plsc.subcore_barrier()
        cp_t = pltpu.make_async_copy(table_sh, table_v, sem_t)
        cp_t.start()
        zf = jnp.zeros((16,), jnp.float32)
        cp_t.wait()
        iot = lax.iota(jnp.int32, 16)

        def compute(slab, g, lenb_vec):
            ids = jnp.minimum(8 * g + iot, L - 1)
            fv = plsc.load_gather(fidx_v, [ids])
            wg = plsc.load_gather(table_v, [fv])
            pos = 8 * g + iot
            wg = jnp.where(pos < lenb_vec, wg, jnp.zeros_like(wg))
            wsps = [jnp.full((16,), wg[r]) for r in range(ROWS)]
            for ch in range(DV // 16):
                base = ch * 16
                accs = [
                    slab[0, pl.ds(16 * (base + dd), 16)] * wsps[0]
                    for dd in range(16)
                ]
                for r in range(1, ROWS):
                    for dd in range(16):
                        accs[dd] = accs[dd] + (
                            slab[r, pl.ds(16 * (base + dd), 16)] * wsps[r])
                for dd in range(16):
                    plsc.addupdate(
                        acc_v.at[pl.ds(16 * (base + dd), 16)], accs[dd])

        def batch_body(bb, carry):
            b = c * BPC + bb
            cp_f = pltpu.make_async_copy(
                fidx_hbm.at[pl.ds(b * L, L)], fidx_v, sem_f)
            cp_f.start()
            for d in range(DV):
                acc_v[pl.ds(16 * d, 16)] = zf
            lenb_vec = plsc.load_gather(
                len_v, [jnp.full((16,), b, jnp.int32)])
            lenb = lenb_vec[0]
            gb = (lenb + ROWS - 1) // ROWS
            cnt = jnp.maximum((gb - sid + NSUB - 1) // NSUB, 0)

            def slab_copy(buf, sem, g):
                return pltpu.make_async_copy(
                    x_hbm.at[b, N - 1, pl.ds(8 * g, 8), :], buf, sem)

            @pl.when(cnt > 0)
            def _():
                slab_copy(slab_a, sem_a, sid).start()
            cp_f.wait()

            def pair_body(jj, carry2):
                g0 = sid + NSUB * 2 * jj
                g1 = g0 + NSUB
                g2 = g1 + NSUB

                @pl.when(2 * jj + 1 < cnt)
                def _():
                    slab_copy(slab_b, sem_b, g1).start()

                slab_copy(slab_a, sem_a, g0).wait()
                compute(slab_a, g0, lenb_vec)

                @pl.when(2 * jj + 2 < cnt)
                def _():
                    slab_copy(slab_a, sem_a, g2).start()

                @pl.when(2 * jj + 1 < cnt)
                def _():
                    slab_copy(slab_b, sem_b, g1).wait()
                    compute(slab_b, g1, lenb_vec)

                return carry2

            lax.fori_loop(0, (cnt + 1) // 2, pair_body, 0)
            pltpu.sync_copy(acc_v, part_sh.at[sid, pl.ds(bb * D, D)])
            return carry

        lax.fori_loop(0, BPC, batch_body, 0)

        plsc.subcore_barrier()

        @pl.when(sid < BPC)
        def _():
            bb = sid
            b = c * BPC + bb
            pltpu.sync_copy(
                part_sh.at[pl.ds(0, ROWS), pl.ds(bb * D, D)], slab_a)
            pltpu.sync_copy(
                part_sh.at[pl.ds(ROWS, ROWS), pl.ds(bb * D, D)], slab_b)
            for d in range(DV):
                acc_v[pl.ds(16 * d, 16)] = zf

            def comb_body(r, cr):
                for d in range(DV):
                    plsc.addupdate(
                        acc_v.at[pl.ds(16 * d, 16)],
                        slab_a[r, pl.ds(16 * d, 16)]
                        + slab_b[r, pl.ds(16 * d, 16)])
                return cr

            lax.fori_loop(0, ROWS, comb_body, 0)
            pltpu.sync_copy(acc_v, out_hbm.at[pl.ds(b * D, D)])

    return k(x, fidx, wflat, lens)


def kernel(input_feature, input_lengths, vq_indices, weight):
    lens = input_lengths.astype(jnp.int32)
    fidx = (vq_indices[..., 0] * G + vq_indices[..., 1]).reshape(-1)
    out = _sc_pool(input_feature, fidx, weight.reshape(-1), lens)
    return out.reshape(B, D)


# speedup vs baseline: 1.8515x; 1.0548x over previous
import functools

import jax
import jax.numpy as jnp
from jax import lax
from jax.experimental import pallas as pl
from jax.experimental.pallas import tpu as pltpu
from jax.experimental.pallas import tpu_sc as plsc

B, N, L, D = 8, 4, 2048, 768
G = 320
NUM_TILES = 32
TOK = B * L
TPT = TOK // NUM_TILES
BL = 512
NJ = L // BL


def _sc_gather(vq_indices, weight, lens):
    mesh = plsc.VectorSubcoreMesh(core_axis_name="c", subcore_axis_name="s")

    @functools.partial(
        pl.kernel,
        out_type=jax.ShapeDtypeStruct((B, NJ, 1, BL), jnp.float32),
        mesh=mesh,
        scratch_types=[
            pltpu.VMEM_SHARED((G, G), jnp.float32),
            pltpu.VMEM((G, G), jnp.float32),
            pltpu.VMEM((TPT,), jnp.int32),
            pltpu.VMEM((TPT,), jnp.float32),
            pltpu.VMEM((8,), jnp.int32),
            pltpu.SemaphoreType.DMA,
            pltpu.SemaphoreType.DMA,
        ],
        compiler_params=pltpu.CompilerParams(
            needs_layout_passes=False, use_tc_tiling_on_sc=False),
    )
    def k(vq_hbm, wt_hbm, len_hbm, w_hbm, table_sh, table_v, idx_v, w_v,
          len_v, sem0, sem1):
        sid = lax.axis_index("s")
        wid = sid * 2 + lax.axis_index("c")
        b = wid // NJ
        jblk = wid % NJ
        l0 = jblk * TPT

        cp1 = pltpu.make_async_copy(
            vq_hbm.at[pl.ds(wid * TPT, TPT)], idx_v, sem1)
        cp1.start()
        pltpu.sync_copy(len_hbm, len_v)

        @pl.when(sid == 0)
        def _():
            pltpu.sync_copy(wt_hbm, table_sh)
        plsc.subcore_barrier()
        cp0 = pltpu.make_async_copy(table_sh, table_v, sem0)
        cp0.start()

        lenb = plsc.load_gather(len_v, [jnp.full((16,), b, jnp.int32)])
        iot = lax.iota(jnp.int32, 16)
        cp1.wait()
        cp0.wait()
        for j in range(TPT // 16):
            rows = j * 16 + iot
            fv = plsc.load_gather(idx_v, [rows])
            i0 = fv // G
            i1 = fv - i0 * G
            wv = plsc.load_gather(table_v, [i0, i1])
            pos = l0 + rows
            wv = jnp.where(pos < lenb, wv, jnp.zeros_like(wv))
            w_v[pl.ds(j * 16, 16)] = wv
        pltpu.sync_copy(w_v, w_hbm.at[b, jblk, 0])

    return k(vq_indices, weight, lens)


def _tc_reduce(x_full, w4, lens):
    def body(lens_ref, w_ref, x_ref, o_ref):
        b = pl.program_id(0)
        j = pl.program_id(1)

        @pl.when((b == 0) & (j == 0))
        def _():
            o_ref[...] = jnp.zeros_like(o_ref)

        @pl.when(j * BL < lens_ref[b])
        def _():
            wv = w_ref[b, j]
            xm = x_ref[0, 0]
            o_ref[pl.ds(b, 1), :] += lax.dot_general(
                wv, xm, (((1,), (0,)), ((), ())),
                preferred_element_type=jnp.float32)

    def x_map(b, j, lens):
        jmax = jnp.maximum((lens[b] + BL - 1) // BL - 1, 0)
        return (b, N - 1, jnp.minimum(j, jmax), 0)

    grid_spec = pltpu.PrefetchScalarGridSpec(
        num_scalar_prefetch=1,
        grid=(B, NJ),
        in_specs=[
            pl.BlockSpec((B, NJ, 1, BL), lambda b, j, lens: (0, 0, 0, 0)),
            pl.BlockSpec((1, 1, BL, D), x_map),
        ],
        out_specs=pl.BlockSpec((B, D), lambda b, j, lens: (0, 0)),
    )
    return pl.pallas_call(
        body,
        grid_spec=grid_spec,
        out_shape=jax.ShapeDtypeStruct((B, D), jnp.float32),
        compiler_params=pltpu.CompilerParams(
            dimension_semantics=("arbitrary", "arbitrary")),
    )(lens, w4, x_full)


def kernel(input_feature, input_lengths, vq_indices, weight):
    lens = input_lengths.astype(jnp.int32)
    fidx = (vq_indices[..., 0] * G + vq_indices[..., 1]).reshape(-1)
    w4 = _sc_gather(fidx, weight, lens)
    return _tc_reduce(input_feature, w4, lens)


# speedup vs baseline: 2.2505x; 1.2155x over previous
import functools

import jax
import jax.numpy as jnp
from jax import lax
from jax.experimental import pallas as pl
from jax.experimental.pallas import tpu as pltpu
from jax.experimental.pallas import tpu_sc as plsc

B, N, L, D = 8, 4, 2048, 768
G = 320
NUM_TILES = 32
TOK = B * L
TPT = TOK // NUM_TILES
BL = 512
NJ = L // BL


def _sc_gather(fidx, wflat, lens):
    mesh = plsc.VectorSubcoreMesh(core_axis_name="c", subcore_axis_name="s")

    @functools.partial(
        pl.kernel,
        out_type=jax.ShapeDtypeStruct((B, NJ, 1, BL), jnp.float32),
        mesh=mesh,
        scratch_types=[
            pltpu.VMEM((TPT,), jnp.int32),
            pltpu.VMEM((TPT,), jnp.float32),
            pltpu.VMEM((16,), jnp.int32),
            pltpu.SemaphoreType.DMA,
            pltpu.SemaphoreType.DMA,
            pltpu.SemaphoreType.DMA,
        ],
        compiler_params=pltpu.CompilerParams(needs_layout_passes=False),
    )
    def k(fidx_hbm, wt_hbm, len_hbm, w_hbm, idx_v, w_v, len_v, sem0, sem1,
          sem2):
        sid = lax.axis_index("s")
        wid = sid * 2 + lax.axis_index("c")
        b = wid // NJ
        jblk = wid % NJ
        l0 = jblk * TPT

        cp1 = pltpu.make_async_copy(
            fidx_hbm.at[pl.ds(wid * TPT, TPT)], idx_v, sem1)
        cp1.start()
        pltpu.make_async_copy(len_hbm, len_v.at[pl.ds(0, 8)], sem2).start()
        cp1.wait()
        for t in range(TPT // 128):
            pltpu.make_async_copy(
                wt_hbm.at[idx_v.at[pl.ds(t * 128, 128)]],
                w_v.at[pl.ds(t * 128, 128)], sem0).start()
        pltpu.make_async_copy(len_hbm, len_v.at[pl.ds(0, 8)], sem2).wait()
        for t in range(TPT // 128):
            pltpu.make_async_copy(
                wt_hbm.at[idx_v.at[pl.ds(t * 128, 128)]],
                w_v.at[pl.ds(t * 128, 128)], sem0).wait()

        lenb = plsc.load_gather(len_v, [jnp.full((16,), b, jnp.int32)])
        iot = lax.iota(jnp.int32, 16)
        for j in range(TPT // 16):
            rows = j * 16 + iot
            wv = w_v[pl.ds(j * 16, 16)]
            pos = l0 + rows
            w_v[pl.ds(j * 16, 16)] = jnp.where(
                pos < lenb, wv, jnp.zeros_like(wv))
        pltpu.sync_copy(w_v, w_hbm.at[b, jblk, 0])

    return k(fidx, wflat, lens)


def _tc_reduce(x_full, w4, lens):
    def body(lens_ref, w_ref, x_ref, o_ref):
        b = pl.program_id(0)
        j = pl.program_id(1)

        @pl.when((b == 0) & (j == 0))
        def _():
            o_ref[...] = jnp.zeros_like(o_ref)

        @pl.when(j * BL < lens_ref[b])
        def _():
            wv = w_ref[b, j]
            xm = x_ref[0, 0]
            o_ref[pl.ds(b, 1), :] += lax.dot_general(
                wv, xm, (((1,), (0,)), ((), ())),
                preferred_element_type=jnp.float32)

    def x_map(b, j, lens):
        jmax = jnp.maximum((lens[b] + BL - 1) // BL - 1, 0)
        return (b, N - 1, jnp.minimum(j, jmax), 0)

    grid_spec = pltpu.PrefetchScalarGridSpec(
        num_scalar_prefetch=1,
        grid=(B, NJ),
        in_specs=[
            pl.BlockSpec((B, NJ, 1, BL), lambda b, j, lens: (0, 0, 0, 0)),
            pl.BlockSpec((1, 1, BL, D), x_map),
        ],
        out_specs=pl.BlockSpec((B, D), lambda b, j, lens: (0, 0)),
    )
    return pl.pallas_call(
        body,
        grid_spec=grid_spec,
        out_shape=jax.ShapeDtypeStruct((B, D), jnp.float32),
        compiler_params=pltpu.CompilerParams(
            dimension_semantics=("arbitrary", "arbitrary")),
    )(lens, w4, x_full)


def kernel(input_feature, input_lengths, vq_indices, weight):
    lens = input_lengths.astype(jnp.int32)
    fidx = (vq_indices[..., 0] * G + vq_indices[..., 1]).reshape(-1)
    w4 = _sc_gather(fidx, weight.reshape(-1), lens)
    return _tc_reduce(input_feature, w4, lens)


# speedup vs baseline: 2.5027x; 1.1120x over previous
import functools

import jax
import jax.numpy as jnp
from jax import lax
from jax.experimental import pallas as pl
from jax.experimental.pallas import tpu as pltpu
from jax.experimental.pallas import tpu_sc as plsc

B, N, L, D = 8, 4, 2048, 768
G = 320
NUM_TILES = 32
TOK = B * L
TPT = TOK // NUM_TILES
BL = 512
NJ = L // BL


def _sc_gather(fidx, wflat, lens):
    mesh = plsc.VectorSubcoreMesh(core_axis_name="c", subcore_axis_name="s")

    @functools.partial(
        pl.kernel,
        out_type=jax.ShapeDtypeStruct((B, NJ, 1, BL), jnp.float32),
        mesh=mesh,
        scratch_types=[
            pltpu.VMEM((TPT,), jnp.int32),
            pltpu.VMEM((TPT,), jnp.float32),
            pltpu.VMEM((16,), jnp.int32),
            pltpu.SemaphoreType.DMA,
            pltpu.SemaphoreType.DMA,
            pltpu.SemaphoreType.DMA,
        ],
        compiler_params=pltpu.CompilerParams(needs_layout_passes=False),
    )
    def k(fidx_hbm, wt_hbm, len_hbm, w_hbm, idx_v, w_v, len_v, sem0, sem1,
          sem2):
        sid = lax.axis_index("s")
        wid = sid * 2 + lax.axis_index("c")
        b = wid // NJ
        jblk = wid % NJ
        l0 = jblk * TPT

        cp1 = pltpu.make_async_copy(
            fidx_hbm.at[pl.ds(wid * TPT, TPT)], idx_v, sem1)
        cp1.start()
        pltpu.make_async_copy(len_hbm, len_v.at[pl.ds(0, 8)], sem2).start()
        cp1.wait()
        for t in range(TPT // 128):
            pltpu.make_async_copy(
                wt_hbm.at[idx_v.at[pl.ds(t * 128, 128)]],
                w_v.at[pl.ds(t * 128, 128)], sem0).start()
        pltpu.make_async_copy(len_hbm, len_v.at[pl.ds(0, 8)], sem2).wait()
        for t in range(TPT // 128):
            pltpu.make_async_copy(
                wt_hbm.at[idx_v.at[pl.ds(t * 128, 128)]],
                w_v.at[pl.ds(t * 128, 128)], sem0).wait()

        lenb = plsc.load_gather(len_v, [jnp.full((16,), b, jnp.int32)])
        iot = lax.iota(jnp.int32, 16)
        for j in range(TPT // 16):
            rows = j * 16 + iot
            wv = w_v[pl.ds(j * 16, 16)]
            pos = l0 + rows
            w_v[pl.ds(j * 16, 16)] = jnp.where(
                pos < lenb, wv, jnp.zeros_like(wv))
        pltpu.sync_copy(w_v, w_hbm.at[b, jblk, 0])

    return k(fidx, wflat, lens)


def _tc_reduce(x_full, w4, lens):
    NBUF = 4

    def body(lens_ref, w_ref, x_ref, o_ref, *scratch):
        bufs = scratch[:NBUF]
        sems = scratch[NBUF:]
        o_ref[...] = jnp.zeros_like(o_ref)
        nb = [(lens_ref[b] + BL - 1) // BL for b in range(B)]

        def slot_copy(s):
            b, j = divmod(s, NJ)
            return pltpu.make_async_copy(
                x_ref.at[b, N - 1, pl.ds(j * BL, BL), :],
                bufs[s % NBUF], sems[s % NBUF])

        for s in range(NBUF):
            b, j = divmod(s, NJ)

            @pl.when(j < nb[b])
            def _(s=s):
                slot_copy(s).start()

        for s in range(B * NJ):
            b, j = divmod(s, NJ)

            @pl.when(j < nb[b])
            def _(s=s, b=b, j=j):
                slot_copy(s).wait()
                o_ref[b:b + 1, :] += lax.dot_general(
                    w_ref[b, j], bufs[s % NBUF][...],
                    (((1,), (0,)), ((), ())),
                    preferred_element_type=jnp.float32)

            s2 = s + NBUF
            if s2 < B * NJ:
                b2, j2 = divmod(s2, NJ)

                @pl.when(j2 < nb[b2])
                def _(s2=s2):
                    slot_copy(s2).start()

    grid_spec = pltpu.PrefetchScalarGridSpec(
        num_scalar_prefetch=1,
        grid=(1,),
        in_specs=[
            pl.BlockSpec((B, NJ, 1, BL), lambda i, lens: (0, 0, 0, 0)),
            pl.BlockSpec(memory_space=pl.ANY),
        ],
        out_specs=pl.BlockSpec((B, D), lambda i, lens: (0, 0)),
        scratch_shapes=(
            [pltpu.VMEM((BL, D), jnp.float32) for _ in range(NBUF)]
            + [pltpu.SemaphoreType.DMA for _ in range(NBUF)]
        ),
    )
    return pl.pallas_call(
        body,
        grid_spec=grid_spec,
        out_shape=jax.ShapeDtypeStruct((B, D), jnp.float32),
    )(lens, w4, x_full)


def kernel(input_feature, input_lengths, vq_indices, weight):
    lens = input_lengths.astype(jnp.int32)
    fidx = (vq_indices[..., 0] * G + vq_indices[..., 1]).reshape(-1)
    w4 = _sc_gather(fidx, weight.reshape(-1), lens)
    return _tc_reduce(input_feature, w4, lens)


# speedup vs baseline: 2.6382x; 1.0542x over previous
import functools

import jax
import jax.numpy as jnp
from jax import lax
from jax.experimental import pallas as pl
from jax.experimental.pallas import tpu as pltpu
from jax.experimental.pallas import tpu_sc as plsc

B, N, L, D = 8, 4, 2048, 768
G = 320
NUM_TILES = 32
TOK = B * L
TPT = TOK // NUM_TILES
BL = 512
NJ = L // BL


def _sc_gather(fidx, wflat):
    mesh = plsc.VectorSubcoreMesh(core_axis_name="c", subcore_axis_name="s")

    @functools.partial(
        pl.kernel,
        out_type=jax.ShapeDtypeStruct((B, NJ, 1, BL), jnp.float32),
        mesh=mesh,
        scratch_types=[
            pltpu.VMEM((TPT,), jnp.int32),
            pltpu.VMEM((TPT,), jnp.float32),
            pltpu.SemaphoreType.DMA,
            pltpu.SemaphoreType.DMA,
        ],
        compiler_params=pltpu.CompilerParams(needs_layout_passes=False),
    )
    def k(fidx_hbm, wt_hbm, w_hbm, idx_v, w_v, sem0, sem1):
        sid = lax.axis_index("s")
        wid = sid * 2 + lax.axis_index("c")
        b = wid // NJ
        jblk = wid % NJ

        cp1 = pltpu.make_async_copy(
            fidx_hbm.at[pl.ds(wid * TPT, TPT)], idx_v, sem1)
        cp1.start()
        cp1.wait()
        for t in range(TPT // 128):
            pltpu.make_async_copy(
                wt_hbm.at[idx_v.at[pl.ds(t * 128, 128)]],
                w_v.at[pl.ds(t * 128, 128)], sem0).start()
        for t in range(TPT // 128):
            pltpu.make_async_copy(
                wt_hbm.at[idx_v.at[pl.ds(t * 128, 128)]],
                w_v.at[pl.ds(t * 128, 128)], sem0).wait()
        pltpu.sync_copy(w_v, w_hbm.at[b, jblk, 0])

    return k(fidx, wflat)


def _tc_reduce(x_full, w4, lens):
    NBUF = 5
    SUB = 64
    NS = BL // SUB

    def body(lens_ref, w_ref, x_ref, o_ref, *scratch):
        bufs = scratch[:NBUF]
        sems = scratch[NBUF:]
        o_ref[...] = jnp.zeros_like(o_ref)
        nb = [(lens_ref[b] + BL - 1) // BL for b in range(B)]
        nbf = [lens_ref[b] // BL for b in range(B)]

        def transfers(s):
            b, j = divmod(s, NJ)
            m = s % NBUF
            full = pltpu.make_async_copy(
                x_ref.at[b, N - 1, pl.ds(j * BL, BL), :], bufs[m], sems[m])
            rem = lens_ref[b] - j * BL
            subs = [
                (k * SUB < rem,
                 pltpu.make_async_copy(
                     x_ref.at[b, N - 1, pl.ds(j * BL + k * SUB, SUB), :],
                     bufs[m].at[pl.ds(k * SUB, SUB), :], sems[m]))
                for k in range(NS)
            ]
            return b, j, full, subs

        def start_slot(s):
            b, j, full, subs = transfers(s)

            @pl.when(j < nbf[b])
            def _():
                full.start()

            @pl.when((j == nbf[b]) & (j < nb[b]))
            def _():
                for ok, cp in subs:
                    @pl.when(ok)
                    def _(cp=cp):
                        cp.start()

        def wait_slot(s):
            b, j, full, subs = transfers(s)

            @pl.when(j < nbf[b])
            def _():
                full.wait()

            @pl.when((j == nbf[b]) & (j < nb[b]))
            def _():
                for ok, cp in subs:
                    @pl.when(ok)
                    def _(cp=cp):
                        cp.wait()

        for s in range(NBUF - 1):
            start_slot(s)

        pos = lax.broadcasted_iota(jnp.int32, (1, BL), 1)
        for s in range(B * NJ):
            b, j = divmod(s, NJ)
            wait_slot(s)
            if s + NBUF - 1 < B * NJ:
                start_slot(s + NBUF - 1)

            @pl.when(j < nb[b])
            def _(b=b, j=j, m=s % NBUF):
                wv = jnp.where(
                    j * BL + pos < lens_ref[b], w_ref[b, j],
                    jnp.zeros((1, BL), jnp.float32))
                o_ref[b:b + 1, :] += lax.dot_general(
                    wv, bufs[m][...], (((1,), (0,)), ((), ())),
                    preferred_element_type=jnp.float32)

    grid_spec = pltpu.PrefetchScalarGridSpec(
        num_scalar_prefetch=1,
        grid=(1,),
        in_specs=[
            pl.BlockSpec((B, NJ, 1, BL), lambda i, lens: (0, 0, 0, 0)),
            pl.BlockSpec(memory_space=pl.ANY),
        ],
        out_specs=pl.BlockSpec((B, D), lambda i, lens: (0, 0)),
        scratch_shapes=(
            [pltpu.VMEM((BL, D), jnp.float32) for _ in range(NBUF)]
            + [pltpu.SemaphoreType.DMA for _ in range(NBUF)]
        ),
    )
    return pl.pallas_call(
        body,
        grid_spec=grid_spec,
        out_shape=jax.ShapeDtypeStruct((B, D), jnp.float32),
    )(lens, w4, x_full)


def kernel(input_feature, input_lengths, vq_indices, weight):
    lens = input_lengths.astype(jnp.int32)
    fidx = (vq_indices[..., 0] * G + vq_indices[..., 1]).reshape(-1)
    w4 = _sc_gather(fidx, weight.reshape(-1))
    return _tc_reduce(input_feature, w4, lens)


# speedup vs baseline: 2.8300x; 1.0727x over previous
import functools

import jax
import jax.numpy as jnp
from jax import lax
from jax.experimental import pallas as pl
from jax.experimental.pallas import tpu as pltpu
from jax.experimental.pallas import tpu_sc as plsc

B, N, L, D = 8, 4, 2048, 768
G = 320
NUM_TILES = 32
TOK = B * L
TPT = TOK // NUM_TILES
BL = 1024
NJ = L // BL


def _sc_gather(fidx, wflat):
    mesh = plsc.VectorSubcoreMesh(core_axis_name="c", subcore_axis_name="s")

    @functools.partial(
        pl.kernel,
        out_type=jax.ShapeDtypeStruct((B, NJ, 1, BL), jnp.float32),
        mesh=mesh,
        scratch_types=[
            pltpu.VMEM((TPT,), jnp.int32),
            pltpu.VMEM((TPT,), jnp.float32),
            pltpu.SemaphoreType.DMA,
            pltpu.SemaphoreType.DMA,
        ],
        compiler_params=pltpu.CompilerParams(needs_layout_passes=False),
    )
    def k(fidx_hbm, wt_hbm, w_hbm, idx_v, w_v, sem0, sem1):
        sid = lax.axis_index("s")
        wid = sid * 2 + lax.axis_index("c")
        spb = L // TPT
        b = wid // spb
        q = wid % spb
        jblk = q // (BL // TPT)
        off = (q % (BL // TPT)) * TPT

        cp1 = pltpu.make_async_copy(
            fidx_hbm.at[pl.ds(wid * TPT, TPT)], idx_v, sem1)
        cp1.start()
        cp1.wait()
        for t in range(TPT // 128):
            pltpu.make_async_copy(
                wt_hbm.at[idx_v.at[pl.ds(t * 128, 128)]],
                w_v.at[pl.ds(t * 128, 128)], sem0).start()
        for t in range(TPT // 128):
            pltpu.make_async_copy(
                wt_hbm.at[idx_v.at[pl.ds(t * 128, 128)]],
                w_v.at[pl.ds(t * 128, 128)], sem0).wait()
        pltpu.sync_copy(w_v, w_hbm.at[b, jblk, 0, pl.ds(off, TPT)])

    return k(fidx, wflat)


def _tc_reduce(x_full, w4, lens):
    NBUF = 5
    SUB = 64
    NS = BL // SUB

    def body(lens_ref, w_ref, x_ref, o_ref, *scratch):
        bufs = scratch[:NBUF]
        sems = scratch[NBUF:]
        o_ref[...] = jnp.zeros_like(o_ref)
        nb = [(lens_ref[b] + BL - 1) // BL for b in range(B)]
        nbf = [lens_ref[b] // BL for b in range(B)]

        def transfers(s):
            b, j = divmod(s, NJ)
            m = s % NBUF
            full = pltpu.make_async_copy(
                x_ref.at[b, N - 1, pl.ds(j * BL, BL), :], bufs[m], sems[m])
            rem = lens_ref[b] - j * BL
            subs = [
                (k * SUB < rem,
                 pltpu.make_async_copy(
                     x_ref.at[b, N - 1, pl.ds(j * BL + k * SUB, SUB), :],
                     bufs[m].at[pl.ds(k * SUB, SUB), :], sems[m]))
                for k in range(NS)
            ]
            return b, j, full, subs

        def start_slot(s):
            b, j, full, subs = transfers(s)

            @pl.when(j < nbf[b])
            def _():
                full.start()

            @pl.when((j == nbf[b]) & (j < nb[b]))
            def _():
                for ok, cp in subs:
                    @pl.when(ok)
                    def _(cp=cp):
                        cp.start()

        def wait_slot(s):
            b, j, full, subs = transfers(s)

            @pl.when(j < nbf[b])
            def _():
                full.wait()

            @pl.when((j == nbf[b]) & (j < nb[b]))
            def _():
                for ok, cp in subs:
                    @pl.when(ok)
                    def _(cp=cp):
                        cp.wait()

        for s in range(NBUF - 1):
            start_slot(s)

        pos = lax.broadcasted_iota(jnp.int32, (1, BL), 1)
        for s in range(B * NJ):
            b, j = divmod(s, NJ)
            wait_slot(s)
            if s + NBUF - 1 < B * NJ:
                start_slot(s + NBUF - 1)

            @pl.when(j < nb[b])
            def _(b=b, j=j, m=s % NBUF):
                wv = jnp.where(
                    j * BL + pos < lens_ref[b], w_ref[b, j],
                    jnp.zeros((1, BL), jnp.float32))
                o_ref[b:b + 1, :] += lax.dot_general(
                    wv, bufs[m][...], (((1,), (0,)), ((), ())),
                    preferred_element_type=jnp.float32)

    grid_spec = pltpu.PrefetchScalarGridSpec(
        num_scalar_prefetch=1,
        grid=(1,),
        in_specs=[
            pl.BlockSpec((B, NJ, 1, BL), lambda i, lens: (0, 0, 0, 0)),
            pl.BlockSpec(memory_space=pl.ANY),
        ],
        out_specs=pl.BlockSpec((B, D), lambda i, lens: (0, 0)),
        scratch_shapes=(
            [pltpu.VMEM((BL, D), jnp.float32) for _ in range(NBUF)]
            + [pltpu.SemaphoreType.DMA for _ in range(NBUF)]
        ),
    )
    return pl.pallas_call(
        body,
        grid_spec=grid_spec,
        out_shape=jax.ShapeDtypeStruct((B, D), jnp.float32),
    )(lens, w4, x_full)


def kernel(input_feature, input_lengths, vq_indices, weight):
    lens = input_lengths.astype(jnp.int32)
    fidx = (vq_indices[..., 0] * G + vq_indices[..., 1]).reshape(-1)
    w4 = _sc_gather(fidx, weight.reshape(-1))
    return _tc_reduce(input_feature, w4, lens)
